# baseline (speedup 1.0000x reference)
import dataclasses
import functools

import jax
import jax.numpy as jnp
from jax import lax
from jax.experimental import pallas as pl
from jax.experimental.pallas import tpu as pltpu
from jax.experimental.pallas import tpu_sc as plsc

NC = 2
NS = 16
L = 16
NW = NC * NS
IDX_CHUNK = 128


@functools.partial(jax.jit, static_argnames=("B", "D"))
def _mf_dot(user_idx, item_idx, user_emb, item_emb, *, B, D):
    bpw = B // NW
    nchunk = bpw // IDX_CHUNK
    mesh = plsc.VectorSubcoreMesh(
        core_axis_name="c", subcore_axis_name="s", num_cores=NC, num_subcores=NS
    )

    cparams = pltpu.CompilerParams(
        needs_layout_passes=False, use_tc_tiling_on_sc=False
    )

    @functools.partial(
        pl.kernel,
        mesh=mesh,
        compiler_params=cparams,
        out_type=jax.ShapeDtypeStruct((B,), jnp.float32),
        scratch_types=[
            pltpu.VMEM((nchunk, IDX_CHUNK), jnp.int32),
            pltpu.VMEM((nchunk, IDX_CHUNK), jnp.int32),
            pltpu.VMEM((bpw, D), jnp.float32),
            pltpu.VMEM((bpw, D), jnp.float32),
            pltpu.VMEM((bpw,), jnp.float32),
            pltpu.SemaphoreType.DMA,
        ],
    )
    def k(uemb_hbm, iemb_hbm, uidx_hbm, iidx_hbm, out_hbm,
          idxu, idxi, urows, vrows, outv, sem):
        wid = lax.axis_index("s") * NC + lax.axis_index("c")
        base = wid * bpw
        pltpu.sync_copy(uidx_hbm.at[wid], idxu)
        pltpu.sync_copy(iidx_hbm.at[wid], idxi)
        copies = []
        for j in range(nchunk):
            dst = urows.at[pl.ds(j * IDX_CHUNK, IDX_CHUNK)]
            copies.append(pltpu.async_copy(uemb_hbm.at[idxu.at[j]], dst, sem))
            dst = vrows.at[pl.ds(j * IDX_CHUNK, IDX_CHUNK)]
            copies.append(pltpu.async_copy(iemb_hbm.at[idxi.at[j]], dst, sem))
        for c in copies:
            c.wait()

        iota = lax.iota(jnp.int32, L)

        @pl.loop(0, bpw // L)
        def _(g):
            row = g * L + iota
            acc = jnp.zeros((L,), jnp.float32)
            for e in range(D):
                col = iota + e
                col = jnp.where(col >= D, col - D, col)
                u = plsc.load_gather(urows, [row, col])
                v = plsc.load_gather(vrows, [row, col])
                acc = acc + u * v
            outv[pl.ds(g * L, L)] = acc

        pltpu.sync_copy(outv, out_hbm.at[pl.ds(base, bpw)])

    return k(user_emb, item_emb, user_idx, item_idx)


def kernel(user, item, user_emb, item_emb):
    B = user.shape[0]
    D = user_emb.shape[1]
    bpw = B // NW
    nchunk = bpw // IDX_CHUNK
    uidx = user.astype(jnp.int32).reshape(NW, nchunk, IDX_CHUNK)
    iidx = item.astype(jnp.int32).reshape(NW, nchunk, IDX_CHUNK)
    return _mf_dot(uidx, iidx, user_emb, item_emb, B=B, D=D)
